# baseline (speedup 1.0000x reference)
import functools

import jax
import jax.numpy as jnp
import numpy as np
from jax import lax
from jax.experimental import pallas as pl
from jax.experimental.pallas import tpu as pltpu
from jax.experimental.pallas import tpu_sc as plsc

B = 128
V = 100000
N_FEAT = 25
TOPK = 10
LANES = 16
CHUNK = 20000
NCHUNKS = V // CHUNK
GROUP = 25
SUB = 5
GROUPS_PER_CHUNK = CHUNK // (GROUP * LANES)
FPAD = 32
OPAD = 48
NEG = np.float32(-1e30)


def _merge_top16(t_asc, x):
    x_asc = lax.sort(x)
    return lax.sort(jnp.maximum(t_asc, lax.rev(x_asc, (0,))))


def _scan_chunk(buf, carry):
    def group_body(g, c):
        t_asc, thr = c
        gbase = g * jnp.int32(GROUP * LANES)
        submax = []
        for s in range(GROUP // SUB):
            m = buf[pl.ds(gbase + jnp.int32(s * SUB * LANES), LANES)]
            for j in range(1, SUB):
                m = jnp.maximum(
                    m, buf[pl.ds(gbase + jnp.int32((s * SUB + j) * LANES),
                                 LANES)])
            submax.append(m)
        m = submax[0]
        for sm in submax[1:]:
            m = jnp.maximum(m, sm)
        hit = jnp.max(m) > thr

        def do_merge(op):
            t, th = op[0], op[1]
            sms = op[2:]
            for s in range(GROUP // SUB):
                sub_hit = jnp.max(sms[s]) > th

                def sub_merge(t_in, s=s):
                    sbase = gbase + jnp.int32(s * SUB * LANES)

                    def mbody(j, t_acc):
                        x = buf[pl.ds(sbase + j * jnp.int32(LANES), LANES)]
                        return _merge_top16(t_acc, x)

                    return lax.fori_loop(jnp.int32(0), jnp.int32(SUB),
                                         mbody, t_in)

                t = lax.cond(sub_hit, sub_merge, lambda t_in: t_in, t)
                th = jnp.min(t)
            return t, th

        return lax.cond(hit, do_merge, lambda op: (op[0], op[1]),
                        (t_asc, thr) + tuple(submax))

    return lax.fori_loop(jnp.int32(0), jnp.int32(GROUPS_PER_CHUNK),
                         group_body, carry)


def _softmax_row(act_v):
    v0 = act_v[pl.ds(0, LANES)]
    v1 = act_v[pl.ds(16, LANES)]
    v2 = act_v[pl.ds(32, LANES)]
    io = lax.iota(jnp.int32, LANES)
    v1 = jnp.where(io < (N_FEAT - LANES), v1, NEG)
    v2 = jnp.where(io < TOPK, v2, NEG)
    mx = jnp.maximum(jnp.maximum(jnp.max(v0), jnp.max(v1)), jnp.max(v2))
    mxv = jnp.full((LANES,), mx, jnp.float32)
    e0 = jnp.exp(v0 - mxv)
    e1 = jnp.exp(v1 - mxv)
    e2 = jnp.exp(v2 - mxv)
    s = jnp.sum(e0) + jnp.sum(e1) + jnp.sum(e2)
    inv = jnp.full((LANES,), np.float32(1.0), jnp.float32) / jnp.full(
        (LANES,), s, jnp.float32)
    act_v[pl.ds(0, LANES)] = e0 * inv
    act_v[pl.ds(16, LANES)] = e1 * inv
    act_v[pl.ds(32, LANES)] = e2 * inv


def _make_sc_call():
    info = plsc.get_sparse_core_info()
    nw = info.num_cores * info.num_subcores
    rows_per_w = B // nw
    total_chunks = rows_per_w * NCHUNKS
    mesh = plsc.VectorSubcoreMesh(core_axis_name="c", subcore_axis_name="s")

    @functools.partial(
        pl.kernel,
        mesh=mesh,
        out_type=jax.ShapeDtypeStruct((B * OPAD,), jnp.float32),
        scratch_types=[
            pltpu.VMEM((CHUNK,), jnp.float32),
            pltpu.VMEM((CHUNK,), jnp.float32),
            pltpu.VMEM((OPAD,), jnp.float32),
            pltpu.SemaphoreType.DMA,
            pltpu.SemaphoreType.DMA,
        ],
        compiler_params=pltpu.CompilerParams(needs_layout_passes=False),
    )
    def sc_topk(item_hbm, feat_hbm, out_hbm, buf0, buf1, act_v, sem0, sem1):
        wid = (lax.axis_index("s") * jnp.int32(info.num_cores)
               + lax.axis_index("c"))
        base_row = wid * jnp.int32(rows_per_w)
        bufs = (buf0, buf1)
        sems = (sem0, sem1)

        def start(k):
            row = base_row + jnp.int32(k // NCHUNKS)
            off = row * jnp.int32(V) + jnp.int32((k % NCHUNKS) * CHUNK)
            return pltpu.async_copy(item_hbm.at[pl.ds(off, CHUNK)],
                                    bufs[k % 2], sems[k % 2])

        init = jnp.full((LANES,), np.float32(-np.inf), jnp.float32)
        pending = start(0)
        carry = (init, np.float32(-np.inf))
        for k in range(total_chunks):
            pending.wait()
            if k + 1 < total_chunks:
                pending = start(k + 1)
            carry = _scan_chunk(bufs[k % 2], carry)
            if k % NCHUNKS == NCHUNKS - 1:
                row = base_row + jnp.int32(k // NCHUNKS)
                pltpu.sync_copy(feat_hbm.at[pl.ds(row * jnp.int32(FPAD),
                                                  FPAD)],
                                act_v.at[pl.ds(0, FPAD)])
                act_v[pl.ds(32, LANES)] = lax.rev(carry[0], (0,))
                _softmax_row(act_v)
                pltpu.sync_copy(act_v,
                                out_hbm.at[pl.ds(row * jnp.int32(OPAD),
                                                 OPAD)])
                carry = (init, np.float32(-np.inf))

    return sc_topk


def kernel(item_scores, feat_scores, cand_item):
    del cand_item
    feat_pad = jnp.pad(feat_scores.astype(jnp.float32),
                       ((0, 0), (0, FPAD - N_FEAT)))
    out = _make_sc_call()(item_scores.astype(jnp.float32).reshape(-1),
                          feat_pad.reshape(-1))
    r = out.reshape(B, OPAD)
    return jnp.concatenate([r[:, :N_FEAT], r[:, 32:32 + TOPK]], axis=1)


# speedup vs baseline: 1.2280x; 1.2280x over previous
import functools

import jax
import jax.numpy as jnp
import numpy as np
from jax import lax
from jax.experimental import pallas as pl
from jax.experimental.pallas import tpu as pltpu
from jax.experimental.pallas import tpu_sc as plsc

B = 128
V = 100000
N_FEAT = 25
TOPK = 10
LANES = 16
SEG = 40
NSEG = V // SEG
SEG_SLOT = 160
NSLOT_ROW = 16 * SEG_SLOT
CHUNK_SEGS = 4
FULL_CHUNKS = 39
FPAD = 32
OPAD = 48
NEG = np.float32(-1e30)
NINF = np.float32(-np.inf)
I32 = jnp.int32


def _merge_top16(t_asc, x):
    x_asc = lax.sort(x)
    return lax.sort(jnp.maximum(t_asc, lax.rev(x_asc, (0,))))


def _softmax_row(act_v):
    v0 = act_v[pl.ds(0, LANES)]
    v1 = act_v[pl.ds(16, LANES)]
    v2 = act_v[pl.ds(32, LANES)]
    io = lax.iota(jnp.int32, LANES)
    v1 = jnp.where(io < (N_FEAT - LANES), v1, NEG)
    v2 = jnp.where(io < TOPK, v2, NEG)
    mx = jnp.maximum(jnp.maximum(jnp.max(v0), jnp.max(v1)), jnp.max(v2))
    mxv = jnp.full((LANES,), mx, jnp.float32)
    e0 = jnp.exp(v0 - mxv)
    e1 = jnp.exp(v1 - mxv)
    e2 = jnp.exp(v2 - mxv)
    s = jnp.sum(e0) + jnp.sum(e1) + jnp.sum(e2)
    inv = jnp.full((LANES,), np.float32(1.0), jnp.float32) / jnp.full(
        (LANES,), s, jnp.float32)
    act_v[pl.ds(0, LANES)] = e0 * inv
    act_v[pl.ds(16, LANES)] = e1 * inv
    act_v[pl.ds(32, LANES)] = e2 * inv


def _make_sc_call():
    info = plsc.get_sparse_core_info()
    mesh = plsc.VectorSubcoreMesh(core_axis_name="c", subcore_axis_name="s")

    @functools.partial(
        pl.kernel,
        mesh=mesh,
        out_type=jax.ShapeDtypeStruct((B * OPAD,), jnp.float32),
        scratch_types=[
            pltpu.VMEM((CHUNK_SEGS * SEG * 128,), jnp.float32),
            pltpu.VMEM((64 * SEG_SLOT,), jnp.float32),
            pltpu.VMEM((4 * NSLOT_ROW,), jnp.float32),
            pltpu.VMEM((SEG * 128,), jnp.float32),
            pltpu.VMEM((SEG * 128,), jnp.float32),
            pltpu.VMEM((OPAD,), jnp.float32),
            pltpu.VMEM_SHARED((64 * NSLOT_ROW,), jnp.float32),
            pltpu.SemaphoreType.DMA,
            pltpu.SemaphoreType.DMA,
        ],
        compiler_params=pltpu.CompilerParams(needs_layout_passes=False),
    )
    def sc_topk(lt_hbm, feat_hbm, out_hbm, bufa, smt, rb, g0, g1, act_v,
                shared, sg0, sg1):
        h = lax.axis_index("c")
        sub = lax.axis_index("s")
        rbase = h * I32(64)
        lane_iota = lax.iota(jnp.int32, LANES)
        segstart = I32(156) * sub + jnp.maximum(sub - I32(12), I32(0))
        nseg = I32(156) + (sub >= I32(12)).astype(jnp.int32)

        def scan_seg_into(seg_in_buf, segslot):
            accs = tuple(jnp.full((LANES,), NINF, jnp.float32)
                         for _ in range(4))

            def col_body(j, acc_t):
                c = seg_in_buf * I32(SEG) + j
                return tuple(
                    jnp.maximum(acc_t[l],
                                bufa[pl.ds(c * I32(128) + rbase
                                           + I32(16 * l), LANES)])
                    for l in range(4))

            accs = lax.fori_loop(I32(0), I32(SEG), col_body, accs)
            for l in range(4):
                fidx = ((I32(l * LANES) + lane_iota) * I32(SEG_SLOT)
                        + jnp.full((LANES,), segslot, jnp.int32))
                plsc.store_scatter(smt, [fidx], accs[l])

        for hole in range(156, SEG_SLOT):
            for l in range(4):
                fidx = ((I32(l * LANES) + lane_iota) * I32(SEG_SLOT)
                        + jnp.full((LANES,), I32(hole), jnp.int32))
                plsc.store_scatter(smt, [fidx],
                                   jnp.full((LANES,), NINF, jnp.float32))

        def chunk_body(k, _):
            col0 = (segstart + k * I32(CHUNK_SEGS)) * I32(SEG)
            pltpu.sync_copy(
                lt_hbm.at[pl.ds(col0 * I32(128), CHUNK_SEGS * SEG * 128)],
                bufa)

            def seg_body(seg, _):
                scan_seg_into(seg, k * I32(CHUNK_SEGS) + seg)
                return jnp.int32(0)

            lax.fori_loop(I32(0), I32(CHUNK_SEGS), seg_body, jnp.int32(0))
            return jnp.int32(0)

        lax.fori_loop(I32(0), I32(FULL_CHUNKS), chunk_body, jnp.int32(0))

        def tail_body(_):
            col0 = (segstart + I32(156)) * I32(SEG)
            pltpu.sync_copy(lt_hbm.at[pl.ds(col0 * I32(128), SEG * 128)],
                            bufa.at[pl.ds(0, SEG * 128)])
            scan_seg_into(I32(0), I32(156))
            return jnp.int32(0)

        lax.cond(nseg > I32(156), tail_body,
                 lambda _: jnp.int32(0), jnp.int32(0))

        def stage_row(rl, _):
            pltpu.sync_copy(
                smt.at[pl.ds(rl * I32(SEG_SLOT), SEG_SLOT)],
                shared.at[pl.ds(rl * I32(NSLOT_ROW) + sub * I32(SEG_SLOT),
                                SEG_SLOT)])
            return jnp.int32(0)

        lax.fori_loop(I32(0), I32(64), stage_row, jnp.int32(0))
        plsc.subcore_barrier()
        pltpu.sync_copy(
            shared.at[pl.ds(sub * I32(4) * I32(NSLOT_ROW), 4 * NSLOT_ROW)],
            rb)

        NVREG = NSLOT_ROW // LANES
        init16 = jnp.full((LANES,), NINF, jnp.float32)

        def row_body(i, _):
            rloc = sub * I32(4) + i
            r_glob = rbase + rloc
            l_row = r_glob % I32(LANES)
            gq = (r_glob // I32(LANES)) * I32(LANES)
            lmask = lane_iota == jnp.full((LANES,), l_row, jnp.int32)
            rb0 = i * I32(NSLOT_ROW)

            def tg_body(g, c):
                t, th = c
                gbase = rb0 + g * I32(16 * LANES)
                m = rb[pl.ds(gbase, LANES)]
                for j in range(1, 16):
                    m = jnp.maximum(
                        m, rb[pl.ds(gbase + I32(j * LANES), LANES)])
                hit = jnp.max(m) > th

                def dm(op):
                    def mb(j, cc):
                        t3, th3 = cc
                        x = rb[pl.ds(gbase + j * I32(LANES), LANES)]
                        t3 = lax.cond(jnp.max(x) > th3,
                                      lambda tt: _merge_top16(tt, x),
                                      lambda tt: tt, t3)
                        return t3, jnp.min(t3)

                    return lax.fori_loop(I32(0), I32(16), mb, op)

                return lax.cond(hit, dm, lambda op: op, (t, th))

            t_asc, thr = lax.fori_loop(I32(0), I32(NVREG // 16), tg_body,
                                       (init16, NINF))
            tstar = thr
            tsv = jnp.full((LANES,), tstar, jnp.float32)

            def issue(parity, seg_id):
                t_sh = seg_id // I32(SEG_SLOT)
                segloc = seg_id % I32(SEG_SLOT)
                colseg = (I32(156) * t_sh
                          + jnp.maximum(t_sh - I32(12), I32(0)) + segloc)
                src = lt_hbm.at[pl.ds(colseg * I32(SEG) * I32(128),
                                      SEG * 128)]

                def into0(_):
                    pltpu.async_copy(src, g0, sg0)
                    return jnp.int32(0)

                def into1(_):
                    pltpu.async_copy(src, g1, sg1)
                    return jnp.int32(0)

                lax.cond(parity == I32(0), into0, into1, jnp.int32(0))

            def merge_slot(parity, te, the):
                def from_slot(gref, sem):
                    def go(op):
                        t_in, th_in = op
                        pltpu.make_async_copy(
                            lt_hbm.at[pl.ds(I32(0), SEG * 128)],
                            gref, sem).wait()

                        def q_body(q, cc):
                            t_q, th_q = cc
                            qb = q * I32(10 * 128) + gq
                            mm = gref[pl.ds(qb, LANES)]
                            for j in range(1, 10):
                                mm = jnp.maximum(
                                    mm, gref[pl.ds(qb + I32(j * 128),
                                                   LANES)])
                            mm = jnp.where(lmask, mm, NEG)
                            ghit = jnp.max(mm) > th_q

                            def gmerge(t_g):
                                def gm_b(j, t_gg):
                                    x = gref[pl.ds(qb + j * I32(128),
                                                   LANES)]
                                    x = jnp.where(lmask, x, NEG)
                                    return _merge_top16(t_gg, x)

                                return lax.fori_loop(I32(0), I32(10),
                                                     gm_b, t_g)

                            t_q = lax.cond(ghit, gmerge,
                                           lambda t_g: t_g, t_q)
                            return t_q, jnp.min(t_q)

                        return lax.fori_loop(I32(0), I32(SEG // 10),
                                             q_body, (t_in, th_in))
                    return go

                return lax.cond(parity == I32(0),
                                from_slot(g0, sg0), from_slot(g1, sg1),
                                (te, the))

            def vreg_body(k, carry):
                hcnt, te, the = carry
                v = rb[pl.ds(rb0 + k * I32(LANES), LANES)]
                idm = jnp.where(v >= tsv, lane_iota, jnp.int32(99))

                def wcond(c):
                    return jnp.min(c[0]) < jnp.int32(99)

                def wbody(c):
                    idm_c, hcnt_c, te_c, the_c = c
                    l = jnp.min(idm_c)
                    seg_id = k * I32(LANES) + l
                    te_c, the_c = lax.cond(
                        hcnt_c > I32(0),
                        lambda op: merge_slot((hcnt_c - I32(1)) % I32(2),
                                              op[0], op[1]),
                        lambda op: op, (te_c, the_c))
                    issue(hcnt_c % I32(2), seg_id)
                    idm_c = jnp.where(
                        lane_iota == jnp.full((LANES,), l, jnp.int32),
                        jnp.int32(99), idm_c)
                    return idm_c, hcnt_c + I32(1), te_c, the_c

                _, hcnt, te, the = lax.while_loop(
                    wcond, wbody, (idm, hcnt, te, the))
                return hcnt, te, the

            hcnt, te, the = lax.fori_loop(
                I32(0), I32(NVREG), vreg_body,
                (jnp.int32(0), init16, NINF))

            te, the = lax.cond(
                hcnt > I32(0),
                lambda op: merge_slot((op[2] - I32(1)) % I32(2),
                                      op[0], op[1]),
                lambda op: (op[0], op[1]), (te, the, hcnt))

            pltpu.sync_copy(feat_hbm.at[pl.ds(r_glob * I32(FPAD), FPAD)],
                            act_v.at[pl.ds(0, FPAD)])
            act_v[pl.ds(32, LANES)] = lax.rev(te, (0,))
            _softmax_row(act_v)
            pltpu.sync_copy(act_v,
                            out_hbm.at[pl.ds(r_glob * I32(OPAD), OPAD)])
            return jnp.int32(0)

        lax.fori_loop(I32(0), I32(4), row_body, jnp.int32(0))

    return sc_topk


def kernel(item_scores, feat_scores, cand_item):
    del cand_item
    lt = item_scores.astype(jnp.float32).T.reshape(-1)
    feat_pad = jnp.pad(feat_scores.astype(jnp.float32),
                       ((0, 0), (0, FPAD - N_FEAT)))
    out = _make_sc_call()(lt, feat_pad.reshape(-1))
    r = out.reshape(B, OPAD)
    return jnp.concatenate([r[:, :N_FEAT], r[:, 32:32 + TOPK]], axis=1)


# speedup vs baseline: 1.3875x; 1.1299x over previous
import functools

import jax
import jax.numpy as jnp
import numpy as np
from jax import lax
from jax.experimental import pallas as pl
from jax.experimental.pallas import tpu as pltpu
from jax.experimental.pallas import tpu_sc as plsc

B = 128
V = 100000
N_FEAT = 25
TOPK = 10
LANES = 16
SEG = 40
NSEG = V // SEG
SEG_SLOT = 160
NSLOT_ROW = 16 * SEG_SLOT
CHUNK_SEGS = 4
FULL_CHUNKS = 39
FPAD = 32
OPAD = 48
NEG = np.float32(-1e30)
NINF = np.float32(-np.inf)
I32 = jnp.int32


def _merge_top16(t_asc, x):
    x_asc = lax.sort(x)
    return lax.sort(jnp.maximum(t_asc, lax.rev(x_asc, (0,))))


def _softmax_row(act_v):
    v0 = act_v[pl.ds(0, LANES)]
    v1 = act_v[pl.ds(16, LANES)]
    v2 = act_v[pl.ds(32, LANES)]
    io = lax.iota(jnp.int32, LANES)
    v1 = jnp.where(io < (N_FEAT - LANES), v1, NEG)
    v2 = jnp.where(io < TOPK, v2, NEG)
    mx = jnp.maximum(jnp.maximum(jnp.max(v0), jnp.max(v1)), jnp.max(v2))
    mxv = jnp.full((LANES,), mx, jnp.float32)
    e0 = jnp.exp(v0 - mxv)
    e1 = jnp.exp(v1 - mxv)
    e2 = jnp.exp(v2 - mxv)
    s = jnp.sum(e0) + jnp.sum(e1) + jnp.sum(e2)
    inv = jnp.full((LANES,), np.float32(1.0), jnp.float32) / jnp.full(
        (LANES,), s, jnp.float32)
    act_v[pl.ds(0, LANES)] = e0 * inv
    act_v[pl.ds(16, LANES)] = e1 * inv
    act_v[pl.ds(32, LANES)] = e2 * inv


def _make_sc_call():
    info = plsc.get_sparse_core_info()
    mesh = plsc.VectorSubcoreMesh(core_axis_name="c", subcore_axis_name="s")

    @functools.partial(
        pl.kernel,
        mesh=mesh,
        out_type=jax.ShapeDtypeStruct((B * OPAD,), jnp.float32),
        scratch_types=[
            pltpu.VMEM((CHUNK_SEGS * SEG * 128,), jnp.float32),
            pltpu.VMEM((CHUNK_SEGS * SEG * 128,), jnp.float32),
            pltpu.VMEM((64 * SEG_SLOT,), jnp.float32),
            pltpu.VMEM((4 * NSLOT_ROW,), jnp.float32),
            pltpu.VMEM((SEG * 128,), jnp.float32),
            pltpu.VMEM((SEG * 128,), jnp.float32),
            pltpu.VMEM((OPAD,), jnp.float32),
            pltpu.VMEM_SHARED((64 * NSLOT_ROW,), jnp.float32),
            pltpu.SemaphoreType.DMA,
            pltpu.SemaphoreType.DMA,
            pltpu.SemaphoreType.DMA,
            pltpu.SemaphoreType.DMA,
        ],
        compiler_params=pltpu.CompilerParams(needs_layout_passes=False),
    )
    def sc_topk(lt_hbm, feat_hbm, out_hbm, bufa, bufb, smt, rb, g0, g1,
                act_v, shared, sg0, sg1, sa0, sa1):
        h = lax.axis_index("c")
        sub = lax.axis_index("s")
        rbase = h * I32(64)
        lane_iota = lax.iota(jnp.int32, LANES)
        segstart = I32(156) * sub + jnp.maximum(sub - I32(12), I32(0))
        nseg = I32(156) + (sub >= I32(12)).astype(jnp.int32)

        CH_WORDS = CHUNK_SEGS * SEG * 128

        def scan_seg_into(buf, seg_in_buf, segslot):
            accs = tuple(jnp.full((LANES,), NINF, jnp.float32)
                         for _ in range(4))

            def col_body(j, acc_t):
                acc = list(acc_t)
                c0 = (seg_in_buf * I32(SEG) + j * I32(4)) * I32(128) + rbase
                for dc in range(4):
                    for l in range(4):
                        acc[l] = jnp.maximum(
                            acc[l], buf[pl.ds(c0 + I32(dc * 128 + 16 * l),
                                              LANES)])
                return tuple(acc)

            accs = lax.fori_loop(I32(0), I32(SEG // 4), col_body, accs)
            for l in range(4):
                fidx = ((I32(l * LANES) + lane_iota) * I32(SEG_SLOT)
                        + jnp.full((LANES,), segslot, jnp.int32))
                plsc.store_scatter(smt, [fidx], accs[l])

        for hole in range(156, SEG_SLOT):
            for l in range(4):
                fidx = ((I32(l * LANES) + lane_iota) * I32(SEG_SLOT)
                        + jnp.full((LANES,), I32(hole), jnp.int32))
                plsc.store_scatter(smt, [fidx],
                                   jnp.full((LANES,), NINF, jnp.float32))

        def a_chunk_off(k):
            return (segstart + k * I32(CHUNK_SEGS)) * I32(SEG) * I32(128)

        pltpu.async_copy(lt_hbm.at[pl.ds(a_chunk_off(I32(0)), CH_WORDS)],
                         bufa, sa0)

        def process(bufx, semx, bufy, semy, k):
            pltpu.make_async_copy(lt_hbm.at[pl.ds(I32(0), CH_WORDS)],
                                  bufx, semx).wait()

            def start_next(_):
                pltpu.async_copy(
                    lt_hbm.at[pl.ds(a_chunk_off(k + I32(1)), CH_WORDS)],
                    bufy, semy)
                return jnp.int32(0)

            lax.cond(k < I32(FULL_CHUNKS - 1), start_next,
                     lambda _: jnp.int32(0), jnp.int32(0))

            def seg_body(seg, _):
                scan_seg_into(bufx, seg, k * I32(CHUNK_SEGS) + seg)
                return jnp.int32(0)

            lax.fori_loop(I32(0), I32(CHUNK_SEGS), seg_body, jnp.int32(0))
            return jnp.int32(0)

        def chunk_body(k, _):
            return lax.cond(k % I32(2) == I32(0),
                            lambda kk: process(bufa, sa0, bufb, sa1, kk),
                            lambda kk: process(bufb, sa1, bufa, sa0, kk),
                            k)

        lax.fori_loop(I32(0), I32(FULL_CHUNKS), chunk_body, jnp.int32(0))

        def tail_body(_):
            col0 = (segstart + I32(156)) * I32(SEG)
            pltpu.sync_copy(lt_hbm.at[pl.ds(col0 * I32(128), SEG * 128)],
                            bufa.at[pl.ds(0, SEG * 128)])
            scan_seg_into(bufa, I32(0), I32(156))
            return jnp.int32(0)

        lax.cond(nseg > I32(156), tail_body,
                 lambda _: jnp.int32(0), jnp.int32(0))

        def stage_row(rl, _):
            pltpu.sync_copy(
                smt.at[pl.ds(rl * I32(SEG_SLOT), SEG_SLOT)],
                shared.at[pl.ds(rl * I32(NSLOT_ROW) + sub * I32(SEG_SLOT),
                                SEG_SLOT)])
            return jnp.int32(0)

        lax.fori_loop(I32(0), I32(64), stage_row, jnp.int32(0))
        plsc.subcore_barrier()
        pltpu.sync_copy(
            shared.at[pl.ds(sub * I32(4) * I32(NSLOT_ROW), 4 * NSLOT_ROW)],
            rb)

        NVREG = NSLOT_ROW // LANES
        init16 = jnp.full((LANES,), NINF, jnp.float32)

        def row_body(i, _):
            rloc = sub * I32(4) + i
            r_glob = rbase + rloc
            l_row = r_glob % I32(LANES)
            gq = (r_glob // I32(LANES)) * I32(LANES)
            lmask = lane_iota == jnp.full((LANES,), l_row, jnp.int32)
            rb0 = i * I32(NSLOT_ROW)

            def tg_body(g, c):
                t, th = c
                gbase = rb0 + g * I32(16 * LANES)
                m = rb[pl.ds(gbase, LANES)]
                for j in range(1, 16):
                    m = jnp.maximum(
                        m, rb[pl.ds(gbase + I32(j * LANES), LANES)])
                hit = jnp.max(m) > th

                def dm(op):
                    def mb(j, cc):
                        t3, th3 = cc
                        x = rb[pl.ds(gbase + j * I32(LANES), LANES)]
                        t3 = lax.cond(jnp.max(x) > th3,
                                      lambda tt: _merge_top16(tt, x),
                                      lambda tt: tt, t3)
                        return t3, jnp.min(t3)

                    return lax.fori_loop(I32(0), I32(16), mb, op)

                return lax.cond(hit, dm, lambda op: op, (t, th))

            t_asc, thr = lax.fori_loop(I32(0), I32(NVREG // 16), tg_body,
                                       (init16, NINF))
            tstar = thr
            tsv = jnp.full((LANES,), tstar, jnp.float32)

            def issue(parity, seg_id):
                t_sh = seg_id // I32(SEG_SLOT)
                segloc = seg_id % I32(SEG_SLOT)
                colseg = (I32(156) * t_sh
                          + jnp.maximum(t_sh - I32(12), I32(0)) + segloc)
                src = lt_hbm.at[pl.ds(colseg * I32(SEG) * I32(128),
                                      SEG * 128)]

                def into0(_):
                    pltpu.async_copy(src, g0, sg0)
                    return jnp.int32(0)

                def into1(_):
                    pltpu.async_copy(src, g1, sg1)
                    return jnp.int32(0)

                lax.cond(parity == I32(0), into0, into1, jnp.int32(0))

            def merge_slot(parity, te, the):
                def from_slot(gref, sem):
                    def go(op):
                        t_in, th_in = op
                        pltpu.make_async_copy(
                            lt_hbm.at[pl.ds(I32(0), SEG * 128)],
                            gref, sem).wait()

                        def q_body(q, cc):
                            t_q, th_q = cc
                            qb = q * I32(10 * 128) + gq
                            mm = gref[pl.ds(qb, LANES)]
                            for j in range(1, 10):
                                mm = jnp.maximum(
                                    mm, gref[pl.ds(qb + I32(j * 128),
                                                   LANES)])
                            mm = jnp.where(lmask, mm, NEG)
                            ghit = jnp.max(mm) > th_q

                            def gmerge(t_g):
                                def gm_b(j, t_gg):
                                    x = gref[pl.ds(qb + j * I32(128),
                                                   LANES)]
                                    x = jnp.where(lmask, x, NEG)
                                    return _merge_top16(t_gg, x)

                                return lax.fori_loop(I32(0), I32(10),
                                                     gm_b, t_g)

                            t_q = lax.cond(ghit, gmerge,
                                           lambda t_g: t_g, t_q)
                            return t_q, jnp.min(t_q)

                        return lax.fori_loop(I32(0), I32(SEG // 10),
                                             q_body, (t_in, th_in))
                    return go

                return lax.cond(parity == I32(0),
                                from_slot(g0, sg0), from_slot(g1, sg1),
                                (te, the))

            def vreg_body(k, carry):
                hcnt, te, the = carry
                v = rb[pl.ds(rb0 + k * I32(LANES), LANES)]
                idm = jnp.where(v >= tsv, lane_iota, jnp.int32(99))

                def wcond(c):
                    return jnp.min(c[0]) < jnp.int32(99)

                def wbody(c):
                    idm_c, hcnt_c, te_c, the_c = c
                    l = jnp.min(idm_c)
                    seg_id = k * I32(LANES) + l
                    te_c, the_c = lax.cond(
                        hcnt_c > I32(0),
                        lambda op: merge_slot((hcnt_c - I32(1)) % I32(2),
                                              op[0], op[1]),
                        lambda op: op, (te_c, the_c))
                    issue(hcnt_c % I32(2), seg_id)
                    idm_c = jnp.where(
                        lane_iota == jnp.full((LANES,), l, jnp.int32),
                        jnp.int32(99), idm_c)
                    return idm_c, hcnt_c + I32(1), te_c, the_c

                _, hcnt, te, the = lax.while_loop(
                    wcond, wbody, (idm, hcnt, te, the))
                return hcnt, te, the

            hcnt, te, the = lax.fori_loop(
                I32(0), I32(NVREG), vreg_body,
                (jnp.int32(0), init16, NINF))

            te, the = lax.cond(
                hcnt > I32(0),
                lambda op: merge_slot((op[2] - I32(1)) % I32(2),
                                      op[0], op[1]),
                lambda op: (op[0], op[1]), (te, the, hcnt))

            pltpu.sync_copy(feat_hbm.at[pl.ds(r_glob * I32(FPAD), FPAD)],
                            act_v.at[pl.ds(0, FPAD)])
            act_v[pl.ds(32, LANES)] = lax.rev(te, (0,))
            _softmax_row(act_v)
            pltpu.sync_copy(act_v,
                            out_hbm.at[pl.ds(r_glob * I32(OPAD), OPAD)])
            return jnp.int32(0)

        lax.fori_loop(I32(0), I32(4), row_body, jnp.int32(0))

    return sc_topk


def kernel(item_scores, feat_scores, cand_item):
    del cand_item
    lt = item_scores.astype(jnp.float32).T.reshape(-1)
    feat_pad = jnp.pad(feat_scores.astype(jnp.float32),
                       ((0, 0), (0, FPAD - N_FEAT)))
    out = _make_sc_call()(lt, feat_pad.reshape(-1))
    r = out.reshape(B, OPAD)
    return jnp.concatenate([r[:, :N_FEAT], r[:, 32:32 + TOPK]], axis=1)


# speedup vs baseline: 1.4155x; 1.0202x over previous
import functools

import jax
import jax.numpy as jnp
import numpy as np
from jax import lax
from jax.experimental import pallas as pl
from jax.experimental.pallas import tpu as pltpu
from jax.experimental.pallas import tpu_sc as plsc

B = 128
V = 100000
N_FEAT = 25
TOPK = 10
LANES = 16
SEG = 40
NSEG = V // SEG
SEG_SLOT = 160
NSLOT_ROW = 16 * SEG_SLOT
CHUNK_SEGS = 4
FULL_CHUNKS = 39
FPAD = 32
OPAD = 48
NEG = np.float32(-1e30)
NINF = np.float32(-np.inf)
I32 = jnp.int32


def _merge_top16(t_asc, x):
    x_asc = lax.sort(x)
    return lax.sort(jnp.maximum(t_asc, lax.rev(x_asc, (0,))))


def _softmax_row(act_v):
    v0 = act_v[pl.ds(0, LANES)]
    v1 = act_v[pl.ds(16, LANES)]
    v2 = act_v[pl.ds(32, LANES)]
    io = lax.iota(jnp.int32, LANES)
    v1 = jnp.where(io < (N_FEAT - LANES), v1, NEG)
    v2 = jnp.where(io < TOPK, v2, NEG)
    mx = jnp.maximum(jnp.maximum(jnp.max(v0), jnp.max(v1)), jnp.max(v2))
    mxv = jnp.full((LANES,), mx, jnp.float32)
    e0 = jnp.exp(v0 - mxv)
    e1 = jnp.exp(v1 - mxv)
    e2 = jnp.exp(v2 - mxv)
    s = jnp.sum(e0) + jnp.sum(e1) + jnp.sum(e2)
    inv = jnp.full((LANES,), np.float32(1.0), jnp.float32) / jnp.full(
        (LANES,), s, jnp.float32)
    act_v[pl.ds(0, LANES)] = e0 * inv
    act_v[pl.ds(16, LANES)] = e1 * inv
    act_v[pl.ds(32, LANES)] = e2 * inv


def _make_sc_call():
    info = plsc.get_sparse_core_info()
    mesh = plsc.VectorSubcoreMesh(core_axis_name="c", subcore_axis_name="s")

    @functools.partial(
        pl.kernel,
        mesh=mesh,
        out_type=jax.ShapeDtypeStruct((B * OPAD,), jnp.float32),
        scratch_types=[
            pltpu.VMEM((CHUNK_SEGS * SEG * 128,), jnp.float32),
            pltpu.VMEM((CHUNK_SEGS * SEG * 128,), jnp.float32),
            pltpu.VMEM((64 * SEG_SLOT,), jnp.float32),
            pltpu.VMEM((4 * NSLOT_ROW,), jnp.float32),
            pltpu.VMEM((SEG * 128,), jnp.float32),
            pltpu.VMEM((SEG * 128,), jnp.float32),
            pltpu.VMEM((OPAD,), jnp.float32),
            pltpu.VMEM_SHARED((64 * NSLOT_ROW,), jnp.float32),
            pltpu.SemaphoreType.DMA,
            pltpu.SemaphoreType.DMA,
            pltpu.SemaphoreType.DMA,
            pltpu.SemaphoreType.DMA,
        ],
        compiler_params=pltpu.CompilerParams(needs_layout_passes=False),
    )
    def sc_topk(lt_hbm, feat_hbm, out_hbm, bufa, bufb, smt, rb, g0, g1,
                act_v, shared, sg0, sg1, sa0, sa1):
        h = lax.axis_index("c")
        sub = lax.axis_index("s")
        rbase = h * I32(64)
        lane_iota = lax.iota(jnp.int32, LANES)
        segstart = I32(156) * sub + jnp.maximum(sub - I32(12), I32(0))
        nseg = I32(156) + (sub >= I32(12)).astype(jnp.int32)

        CH_WORDS = CHUNK_SEGS * SEG * 128

        def scan_seg_into(buf, seg_in_buf, segslot):
            accs = tuple(jnp.full((LANES,), NINF, jnp.float32)
                         for _ in range(4))

            def col_body(j, acc_t):
                acc = list(acc_t)
                c0 = (seg_in_buf * I32(SEG) + j * I32(4)) * I32(128) + rbase
                for dc in range(4):
                    for l in range(4):
                        acc[l] = jnp.maximum(
                            acc[l], buf[pl.ds(c0 + I32(dc * 128 + 16 * l),
                                              LANES)])
                return tuple(acc)

            accs = lax.fori_loop(I32(0), I32(SEG // 4), col_body, accs)
            for l in range(4):
                fidx = ((I32(l * LANES) + lane_iota) * I32(SEG_SLOT)
                        + jnp.full((LANES,), segslot, jnp.int32))
                plsc.store_scatter(smt, [fidx], accs[l])

        for hole in range(156, SEG_SLOT):
            for l in range(4):
                fidx = ((I32(l * LANES) + lane_iota) * I32(SEG_SLOT)
                        + jnp.full((LANES,), I32(hole), jnp.int32))
                plsc.store_scatter(smt, [fidx],
                                   jnp.full((LANES,), NINF, jnp.float32))

        def a_chunk_off(k):
            return (segstart + k * I32(CHUNK_SEGS)) * I32(SEG) * I32(128)

        pltpu.async_copy(lt_hbm.at[pl.ds(a_chunk_off(I32(0)), CH_WORDS)],
                         bufa, sa0)

        def process(bufx, semx, bufy, semy, k):
            pltpu.make_async_copy(lt_hbm.at[pl.ds(I32(0), CH_WORDS)],
                                  bufx, semx).wait()

            def start_next(_):
                pltpu.async_copy(
                    lt_hbm.at[pl.ds(a_chunk_off(k + I32(1)), CH_WORDS)],
                    bufy, semy)
                return jnp.int32(0)

            lax.cond(k < I32(FULL_CHUNKS - 1), start_next,
                     lambda _: jnp.int32(0), jnp.int32(0))

            def seg_body(seg, _):
                scan_seg_into(bufx, seg, k * I32(CHUNK_SEGS) + seg)
                return jnp.int32(0)

            lax.fori_loop(I32(0), I32(CHUNK_SEGS), seg_body, jnp.int32(0))
            return jnp.int32(0)

        def chunk_body(k, _):
            return lax.cond(k % I32(2) == I32(0),
                            lambda kk: process(bufa, sa0, bufb, sa1, kk),
                            lambda kk: process(bufb, sa1, bufa, sa0, kk),
                            k)

        lax.fori_loop(I32(0), I32(FULL_CHUNKS), chunk_body, jnp.int32(0))

        def tail_body(_):
            col0 = (segstart + I32(156)) * I32(SEG)
            pltpu.sync_copy(lt_hbm.at[pl.ds(col0 * I32(128), SEG * 128)],
                            bufa.at[pl.ds(0, SEG * 128)])
            scan_seg_into(bufa, I32(0), I32(156))
            return jnp.int32(0)

        lax.cond(nseg > I32(156), tail_body,
                 lambda _: jnp.int32(0), jnp.int32(0))

        pltpu.sync_copy(smt,
                        shared.at[pl.ds(sub * I32(64 * SEG_SLOT),
                                        64 * SEG_SLOT)])
        plsc.subcore_barrier()

        def collect(tsh, _):
            pltpu.sync_copy(
                shared.at[pl.ds(tsh * I32(64 * SEG_SLOT)
                                + sub * I32(4 * SEG_SLOT), 4 * SEG_SLOT)],
                rb.at[pl.ds(tsh * I32(4 * SEG_SLOT), 4 * SEG_SLOT)])
            return jnp.int32(0)

        lax.fori_loop(I32(0), I32(16), collect, jnp.int32(0))

        NVREG = NSLOT_ROW // LANES
        init16 = jnp.full((LANES,), NINF, jnp.float32)

        def row_body(i, _):
            rloc = sub * I32(4) + i
            r_glob = rbase + rloc
            l_row = r_glob % I32(LANES)
            gq = (r_glob // I32(LANES)) * I32(LANES)
            lmask = lane_iota == jnp.full((LANES,), l_row, jnp.int32)
            irow0 = i * I32(SEG_SLOT)

            def tg_body(g, c):
                t, th = c
                gbase = g * I32(4 * SEG_SLOT) + irow0
                m = rb[pl.ds(gbase, LANES)]
                for j in range(1, 10):
                    m = jnp.maximum(
                        m, rb[pl.ds(gbase + I32(j * LANES), LANES)])
                hit = jnp.max(m) > th

                def dm(op):
                    def mb(j, cc):
                        t3, th3 = cc
                        x = rb[pl.ds(gbase + j * I32(LANES), LANES)]
                        t3 = lax.cond(jnp.max(x) > th3,
                                      lambda tt: _merge_top16(tt, x),
                                      lambda tt: tt, t3)
                        return t3, jnp.min(t3)

                    return lax.fori_loop(I32(0), I32(10), mb, op)

                return lax.cond(hit, dm, lambda op: op, (t, th))

            t_asc, thr = lax.fori_loop(I32(0), I32(16), tg_body,
                                       (init16, NINF))
            tstar = thr
            tsv = jnp.full((LANES,), tstar, jnp.float32)

            def issue(parity, seg_id):
                t_sh = seg_id // I32(SEG_SLOT)
                segloc = seg_id % I32(SEG_SLOT)
                colseg = (I32(156) * t_sh
                          + jnp.maximum(t_sh - I32(12), I32(0)) + segloc)
                src = lt_hbm.at[pl.ds(colseg * I32(SEG) * I32(128),
                                      SEG * 128)]

                def into0(_):
                    pltpu.async_copy(src, g0, sg0)
                    return jnp.int32(0)

                def into1(_):
                    pltpu.async_copy(src, g1, sg1)
                    return jnp.int32(0)

                lax.cond(parity == I32(0), into0, into1, jnp.int32(0))

            def merge_slot(parity, te, the):
                def from_slot(gref, sem):
                    def go(op):
                        t_in, th_in = op
                        pltpu.make_async_copy(
                            lt_hbm.at[pl.ds(I32(0), SEG * 128)],
                            gref, sem).wait()

                        def q_body(q, cc):
                            t_q, th_q = cc
                            qb = q * I32(10 * 128) + gq
                            mm = gref[pl.ds(qb, LANES)]
                            for j in range(1, 10):
                                mm = jnp.maximum(
                                    mm, gref[pl.ds(qb + I32(j * 128),
                                                   LANES)])
                            mm = jnp.where(lmask, mm, NEG)
                            ghit = jnp.max(mm) > th_q

                            def gmerge(t_g):
                                def gm_b(j, t_gg):
                                    x = gref[pl.ds(qb + j * I32(128),
                                                   LANES)]
                                    x = jnp.where(lmask, x, NEG)
                                    return _merge_top16(t_gg, x)

                                return lax.fori_loop(I32(0), I32(10),
                                                     gm_b, t_g)

                            t_q = lax.cond(ghit, gmerge,
                                           lambda t_g: t_g, t_q)
                            return t_q, jnp.min(t_q)

                        return lax.fori_loop(I32(0), I32(SEG // 10),
                                             q_body, (t_in, th_in))
                    return go

                return lax.cond(parity == I32(0),
                                from_slot(g0, sg0), from_slot(g1, sg1),
                                (te, the))

            def vreg_body(k5, carry):
                wb = ((k5 // I32(2)) * I32(4 * SEG_SLOT) + irow0
                      + (k5 % I32(2)) * I32(5 * LANES))
                vs = [rb[pl.ds(wb + I32(j * LANES), LANES)]
                      for j in range(5)]
                m5 = vs[0]
                for x in vs[1:]:
                    m5 = jnp.maximum(m5, x)
                bhit = jnp.max(m5) >= tstar

                def scan_batch(cin):
                    def per_vreg(j, cc):
                        hcnt_c, te_c, the_c = cc
                        v = rb[pl.ds(wb + j * I32(LANES), LANES)]
                        idm = jnp.where(v >= tsv, lane_iota, jnp.int32(99))

                        def wcond(c):
                            return jnp.min(c[0]) < jnp.int32(99)

                        def wbody(c):
                            idm_w, hcnt_w, te_w, the_w = c
                            l = jnp.min(idm_w)
                            seg_id = ((k5 // I32(2)) * I32(SEG_SLOT)
                                      + ((k5 % I32(2)) * I32(5) + j)
                                      * I32(LANES) + l)
                            te_w, the_w = lax.cond(
                                hcnt_w > I32(0),
                                lambda op: merge_slot(
                                    (hcnt_w - I32(1)) % I32(2),
                                    op[0], op[1]),
                                lambda op: op, (te_w, the_w))
                            issue(hcnt_w % I32(2), seg_id)
                            idm_w = jnp.where(
                                lane_iota == jnp.full((LANES,), l,
                                                      jnp.int32),
                                jnp.int32(99), idm_w)
                            return idm_w, hcnt_w + I32(1), te_w, the_w

                        _, hcnt_c, te_c, the_c = lax.while_loop(
                            wcond, wbody, (idm, hcnt_c, te_c, the_c))
                        return hcnt_c, te_c, the_c

                    return lax.fori_loop(I32(0), I32(5), per_vreg, cin)

                return lax.cond(bhit, scan_batch, lambda cin: cin, carry)

            hcnt, te, the = lax.fori_loop(
                I32(0), I32(32), vreg_body,
                (jnp.int32(0), init16, NINF))

            te, the = lax.cond(
                hcnt > I32(0),
                lambda op: merge_slot((op[2] - I32(1)) % I32(2),
                                      op[0], op[1]),
                lambda op: (op[0], op[1]), (te, the, hcnt))

            pltpu.sync_copy(feat_hbm.at[pl.ds(r_glob * I32(FPAD), FPAD)],
                            act_v.at[pl.ds(0, FPAD)])
            act_v[pl.ds(32, LANES)] = lax.rev(te, (0,))
            _softmax_row(act_v)
            pltpu.sync_copy(act_v,
                            out_hbm.at[pl.ds(r_glob * I32(OPAD), OPAD)])
            return jnp.int32(0)

        lax.fori_loop(I32(0), I32(4), row_body, jnp.int32(0))

    return sc_topk


def kernel(item_scores, feat_scores, cand_item):
    del cand_item
    lt = item_scores.astype(jnp.float32).T.reshape(-1)
    feat_pad = jnp.pad(feat_scores.astype(jnp.float32),
                       ((0, 0), (0, FPAD - N_FEAT)))
    out = _make_sc_call()(lt, feat_pad.reshape(-1))
    r = out.reshape(B, OPAD)
    return jnp.concatenate([r[:, :N_FEAT], r[:, 32:32 + TOPK]], axis=1)


# speedup vs baseline: 1.4820x; 1.0470x over previous
import functools

import jax
import jax.numpy as jnp
import numpy as np
from jax import lax
from jax.experimental import pallas as pl
from jax.experimental.pallas import tpu as pltpu
from jax.experimental.pallas import tpu_sc as plsc

B = 128
V = 100000
N_FEAT = 25
TOPK = 10
LANES = 16
SEG = 40
NSEG = V // SEG
SEG_SLOT = 160
NSLOT_ROW = 16 * SEG_SLOT
CHUNK_SEGS = 8
FULL_CHUNKS = 19
FPAD = 32
OPAD = 48
NEG = np.float32(-1e30)
NINF = np.float32(-np.inf)
I32 = jnp.int32


def _merge_top16(t_asc, x):
    x_asc = lax.sort(x)
    return lax.sort(jnp.maximum(t_asc, lax.rev(x_asc, (0,))))


def _softmax_row(act_v):
    v0 = act_v[pl.ds(0, LANES)]
    v1 = act_v[pl.ds(16, LANES)]
    v2 = act_v[pl.ds(32, LANES)]
    io = lax.iota(jnp.int32, LANES)
    v1 = jnp.where(io < (N_FEAT - LANES), v1, NEG)
    v2 = jnp.where(io < TOPK, v2, NEG)
    mx = jnp.maximum(jnp.maximum(jnp.max(v0), jnp.max(v1)), jnp.max(v2))
    mxv = jnp.full((LANES,), mx, jnp.float32)
    e0 = jnp.exp(v0 - mxv)
    e1 = jnp.exp(v1 - mxv)
    e2 = jnp.exp(v2 - mxv)
    s = jnp.sum(e0) + jnp.sum(e1) + jnp.sum(e2)
    inv = jnp.full((LANES,), np.float32(1.0), jnp.float32) / jnp.full(
        (LANES,), s, jnp.float32)
    act_v[pl.ds(0, LANES)] = e0 * inv
    act_v[pl.ds(16, LANES)] = e1 * inv
    act_v[pl.ds(32, LANES)] = e2 * inv


def _make_sc_call():
    info = plsc.get_sparse_core_info()
    mesh = plsc.VectorSubcoreMesh(core_axis_name="c", subcore_axis_name="s")

    @functools.partial(
        pl.kernel,
        mesh=mesh,
        out_type=jax.ShapeDtypeStruct((B * OPAD,), jnp.float32),
        scratch_types=[
            pltpu.VMEM((CHUNK_SEGS * SEG * 128,), jnp.float32),
            pltpu.VMEM((CHUNK_SEGS * SEG * 128,), jnp.float32),
            pltpu.VMEM((64 * SEG_SLOT,), jnp.float32),
            pltpu.VMEM((4 * NSLOT_ROW,), jnp.float32),
            pltpu.VMEM((SEG * 128,), jnp.float32),
            pltpu.VMEM((SEG * 128,), jnp.float32),
            pltpu.VMEM((OPAD,), jnp.float32),
            pltpu.VMEM_SHARED((64 * NSLOT_ROW,), jnp.float32),
            pltpu.SemaphoreType.DMA,
            pltpu.SemaphoreType.DMA,
            pltpu.SemaphoreType.DMA,
            pltpu.SemaphoreType.DMA,
        ],
        compiler_params=pltpu.CompilerParams(needs_layout_passes=False),
    )
    def sc_topk(lt_hbm, feat_hbm, out_hbm, bufa, bufb, smt, rb, g0, g1,
                act_v, shared, sg0, sg1, sa0, sa1):
        h = lax.axis_index("c")
        sub = lax.axis_index("s")
        rbase = h * I32(64)
        lane_iota = lax.iota(jnp.int32, LANES)
        segstart = I32(156) * sub + jnp.maximum(sub - I32(12), I32(0))
        nseg = I32(156) + (sub >= I32(12)).astype(jnp.int32)

        CH_WORDS = CHUNK_SEGS * SEG * 128

        def scan_seg_into(buf, seg_in_buf, segslot):
            accs = tuple(jnp.full((LANES,), NINF, jnp.float32)
                         for _ in range(4))

            def col_body(j, acc_t):
                acc = list(acc_t)
                c0 = (seg_in_buf * I32(SEG) + j * I32(4)) * I32(128) + rbase
                for dc in range(4):
                    for l in range(4):
                        acc[l] = jnp.maximum(
                            acc[l], buf[pl.ds(c0 + I32(dc * 128 + 16 * l),
                                              LANES)])
                return tuple(acc)

            accs = lax.fori_loop(I32(0), I32(SEG // 4), col_body, accs)
            for l in range(4):
                fidx = ((I32(l * LANES) + lane_iota) * I32(SEG_SLOT)
                        + jnp.full((LANES,), segslot, jnp.int32))
                plsc.store_scatter(smt, [fidx], accs[l])

        for hole in range(156, SEG_SLOT):
            for l in range(4):
                fidx = ((I32(l * LANES) + lane_iota) * I32(SEG_SLOT)
                        + jnp.full((LANES,), I32(hole), jnp.int32))
                plsc.store_scatter(smt, [fidx],
                                   jnp.full((LANES,), NINF, jnp.float32))

        def a_chunk_off(k):
            return (segstart + k * I32(CHUNK_SEGS)) * I32(SEG) * I32(128)

        pltpu.async_copy(lt_hbm.at[pl.ds(a_chunk_off(I32(0)), CH_WORDS)],
                         bufa, sa0)

        def process(bufx, semx, bufy, semy, k):
            pltpu.make_async_copy(lt_hbm.at[pl.ds(I32(0), CH_WORDS)],
                                  bufx, semx).wait()

            def start_next(_):
                pltpu.async_copy(
                    lt_hbm.at[pl.ds(a_chunk_off(k + I32(1)), CH_WORDS)],
                    bufy, semy)
                return jnp.int32(0)

            lax.cond(k < I32(FULL_CHUNKS - 1), start_next,
                     lambda _: jnp.int32(0), jnp.int32(0))

            def seg_body(seg, _):
                scan_seg_into(bufx, seg, k * I32(CHUNK_SEGS) + seg)
                return jnp.int32(0)

            lax.fori_loop(I32(0), I32(CHUNK_SEGS), seg_body, jnp.int32(0))
            return jnp.int32(0)

        def chunk_body(k, _):
            return lax.cond(k % I32(2) == I32(0),
                            lambda kk: process(bufa, sa0, bufb, sa1, kk),
                            lambda kk: process(bufb, sa1, bufa, sa0, kk),
                            k)

        lax.fori_loop(I32(0), I32(FULL_CHUNKS), chunk_body, jnp.int32(0))

        def tail_seg(s, _):
            segslot = I32(19 * CHUNK_SEGS) + s
            col0 = (segstart + segslot) * I32(SEG)
            pltpu.sync_copy(lt_hbm.at[pl.ds(col0 * I32(128), SEG * 128)],
                            bufa.at[pl.ds(0, SEG * 128)])
            scan_seg_into(bufa, I32(0), segslot)
            return jnp.int32(0)

        lax.fori_loop(I32(0), nseg - I32(19 * CHUNK_SEGS), tail_seg,
                      jnp.int32(0))

        pltpu.sync_copy(smt,
                        shared.at[pl.ds(sub * I32(64 * SEG_SLOT),
                                        64 * SEG_SLOT)])
        plsc.subcore_barrier()

        def collect(tsh, _):
            pltpu.sync_copy(
                shared.at[pl.ds(tsh * I32(64 * SEG_SLOT)
                                + sub * I32(4 * SEG_SLOT), 4 * SEG_SLOT)],
                rb.at[pl.ds(tsh * I32(4 * SEG_SLOT), 4 * SEG_SLOT)])
            return jnp.int32(0)

        lax.fori_loop(I32(0), I32(16), collect, jnp.int32(0))

        NVREG = NSLOT_ROW // LANES
        init16 = jnp.full((LANES,), NINF, jnp.float32)

        def row_body(i, _):
            rloc = sub * I32(4) + i
            r_glob = rbase + rloc
            l_row = r_glob % I32(LANES)
            gq = (r_glob // I32(LANES)) * I32(LANES)
            lmask = lane_iota == jnp.full((LANES,), l_row, jnp.int32)
            irow0 = i * I32(SEG_SLOT)

            def tg_body(g, c):
                t, th = c
                gbase = g * I32(4 * SEG_SLOT) + irow0
                m = rb[pl.ds(gbase, LANES)]
                for j in range(1, 10):
                    m = jnp.maximum(
                        m, rb[pl.ds(gbase + I32(j * LANES), LANES)])
                hit = jnp.max(m) > th

                def dm(op):
                    def mb(j, cc):
                        t3, th3 = cc
                        x = rb[pl.ds(gbase + j * I32(LANES), LANES)]
                        t3 = lax.cond(jnp.max(x) > th3,
                                      lambda tt: _merge_top16(tt, x),
                                      lambda tt: tt, t3)
                        return t3, jnp.min(t3)

                    return lax.fori_loop(I32(0), I32(10), mb, op)

                return lax.cond(hit, dm, lambda op: op, (t, th))

            t_asc, thr = lax.fori_loop(I32(0), I32(16), tg_body,
                                       (init16, NINF))
            tstar = thr
            tsv = jnp.full((LANES,), tstar, jnp.float32)

            def issue(parity, seg_id):
                t_sh = seg_id // I32(SEG_SLOT)
                segloc = seg_id % I32(SEG_SLOT)
                colseg = (I32(156) * t_sh
                          + jnp.maximum(t_sh - I32(12), I32(0)) + segloc)
                src = lt_hbm.at[pl.ds(colseg * I32(SEG) * I32(128),
                                      SEG * 128)]

                def into0(_):
                    pltpu.async_copy(src, g0, sg0)
                    return jnp.int32(0)

                def into1(_):
                    pltpu.async_copy(src, g1, sg1)
                    return jnp.int32(0)

                lax.cond(parity == I32(0), into0, into1, jnp.int32(0))

            def merge_slot(parity, te, the):
                def from_slot(gref, sem):
                    def go(op):
                        t_in, th_in = op
                        pltpu.make_async_copy(
                            lt_hbm.at[pl.ds(I32(0), SEG * 128)],
                            gref, sem).wait()

                        def q_body(q, cc):
                            t_q, th_q = cc
                            qb = q * I32(10 * 128) + gq
                            mm = gref[pl.ds(qb, LANES)]
                            for j in range(1, 10):
                                mm = jnp.maximum(
                                    mm, gref[pl.ds(qb + I32(j * 128),
                                                   LANES)])
                            mm = jnp.where(lmask, mm, NEG)
                            ghit = jnp.max(mm) > th_q

                            def gmerge(t_g):
                                def gm_b(j, t_gg):
                                    x = gref[pl.ds(qb + j * I32(128),
                                                   LANES)]
                                    x = jnp.where(lmask, x, NEG)
                                    return _merge_top16(t_gg, x)

                                return lax.fori_loop(I32(0), I32(10),
                                                     gm_b, t_g)

                            t_q = lax.cond(ghit, gmerge,
                                           lambda t_g: t_g, t_q)
                            return t_q, jnp.min(t_q)

                        return lax.fori_loop(I32(0), I32(SEG // 10),
                                             q_body, (t_in, th_in))
                    return go

                return lax.cond(parity == I32(0),
                                from_slot(g0, sg0), from_slot(g1, sg1),
                                (te, the))

            def vreg_body(k5, carry):
                wb = ((k5 // I32(2)) * I32(4 * SEG_SLOT) + irow0
                      + (k5 % I32(2)) * I32(5 * LANES))
                vs = [rb[pl.ds(wb + I32(j * LANES), LANES)]
                      for j in range(5)]
                m5 = vs[0]
                for x in vs[1:]:
                    m5 = jnp.maximum(m5, x)
                bhit = jnp.max(m5) >= tstar

                def scan_batch(cin):
                    def per_vreg(j, cc):
                        hcnt_c, te_c, the_c = cc
                        v = rb[pl.ds(wb + j * I32(LANES), LANES)]
                        idm = jnp.where(v >= tsv, lane_iota, jnp.int32(99))

                        def wcond(c):
                            return jnp.min(c[0]) < jnp.int32(99)

                        def wbody(c):
                            idm_w, hcnt_w, te_w, the_w = c
                            l = jnp.min(idm_w)
                            seg_id = ((k5 // I32(2)) * I32(SEG_SLOT)
                                      + ((k5 % I32(2)) * I32(5) + j)
                                      * I32(LANES) + l)
                            te_w, the_w = lax.cond(
                                hcnt_w > I32(0),
                                lambda op: merge_slot(
                                    (hcnt_w - I32(1)) % I32(2),
                                    op[0], op[1]),
                                lambda op: op, (te_w, the_w))
                            issue(hcnt_w % I32(2), seg_id)
                            idm_w = jnp.where(
                                lane_iota == jnp.full((LANES,), l,
                                                      jnp.int32),
                                jnp.int32(99), idm_w)
                            return idm_w, hcnt_w + I32(1), te_w, the_w

                        _, hcnt_c, te_c, the_c = lax.while_loop(
                            wcond, wbody, (idm, hcnt_c, te_c, the_c))
                        return hcnt_c, te_c, the_c

                    return lax.fori_loop(I32(0), I32(5), per_vreg, cin)

                return lax.cond(bhit, scan_batch, lambda cin: cin, carry)

            hcnt, te, the = lax.fori_loop(
                I32(0), I32(32), vreg_body,
                (jnp.int32(0), init16, NINF))

            te, the = lax.cond(
                hcnt > I32(0),
                lambda op: merge_slot((op[2] - I32(1)) % I32(2),
                                      op[0], op[1]),
                lambda op: (op[0], op[1]), (te, the, hcnt))

            pltpu.sync_copy(feat_hbm.at[pl.ds(r_glob * I32(FPAD), FPAD)],
                            act_v.at[pl.ds(0, FPAD)])
            act_v[pl.ds(32, LANES)] = lax.rev(te, (0,))
            _softmax_row(act_v)
            pltpu.sync_copy(act_v,
                            out_hbm.at[pl.ds(r_glob * I32(OPAD), OPAD)])
            return jnp.int32(0)

        lax.fori_loop(I32(0), I32(4), row_body, jnp.int32(0))

    return sc_topk


def kernel(item_scores, feat_scores, cand_item):
    del cand_item
    lt = item_scores.astype(jnp.float32).T.reshape(-1)
    feat_pad = jnp.pad(feat_scores.astype(jnp.float32),
                       ((0, 0), (0, FPAD - N_FEAT)))
    out = _make_sc_call()(lt, feat_pad.reshape(-1))
    r = out.reshape(B, OPAD)
    return jnp.concatenate([r[:, :N_FEAT], r[:, 32:32 + TOPK]], axis=1)
